# baseline (speedup 1.0000x reference)
import jax
import jax.numpy as jnp
from jax.experimental import pallas as pl

G = 128


def _seg_softmax(scores, seg, num):
    m = jax.ops.segment_max(scores, seg, num_segments=num)
    m = jnp.where(jnp.isfinite(m), m, 0.0)
    ex = jnp.exp(scores - m[seg])
    ssum = jax.ops.segment_sum(ex, seg, num_segments=num)
    return ex / (ssum[seg] + 1e-16)


def _gat(x, src, dst, W, a_s, a_d, b, n):
    h = x @ W
    alpha_src = h @ a_s
    alpha_dst = h @ a_d
    e = jax.nn.leaky_relu(alpha_src[src] + alpha_dst[dst], negative_slope=0.2)
    coef = _seg_softmax(e, dst, n)
    out = jax.ops.segment_sum(coef[:, None] * h[src], dst, num_segments=n)
    return out + b


def _gcn(x, src, dst, W, b, n):
    h = x @ W
    deg = jax.ops.segment_sum(jnp.ones(src.shape[0], dtype=h.dtype), dst, num_segments=n)
    dis = jnp.where(deg > 0, 1.0 / jnp.sqrt(deg), 0.0)
    norm = dis[src] * dis[dst]
    out = jax.ops.segment_sum(norm[:, None] * h[src], dst, num_segments=n)
    return out + b


def _softmax_aggr(x, batch, t, g):
    sc = x * t
    m = jax.ops.segment_max(sc, batch, num_segments=g)
    m = jnp.where(jnp.isfinite(m), m, 0.0)
    ex = jnp.exp(sc - m[batch])
    ssum = jax.ops.segment_sum(ex, batch, num_segments=g)
    alpha = ex / (ssum[batch] + 1e-16)
    return jax.ops.segment_sum(alpha * x, batch, num_segments=g)


def _pelu(v):
    return jnp.where(v > 0, v, jnp.exp(jnp.minimum(v, 0.0)) - 1.0)


def _mlp_body(h_ref, Wl1_ref, bl1_ref, Wl2_ref, bl2_ref, o_ref):
    h = h_ref[...]
    h = _pelu(h @ Wl1_ref[...] + bl1_ref[...])
    h = h @ Wl2_ref[...] + bl2_ref[...]
    o_ref[...] = jax.nn.log_softmax(h, axis=1)


def kernel(x, edge_index, batch, W1, a_src1, a_dst1, b1, W2, a_src2, a_dst2, b2, W3, a_src3, a_dst3, b3, Wg1, bg1, Wg2, bg2, t, Wl1, bl1, Wl2, bl2):
    n = x.shape[0]
    loop = jnp.arange(n, dtype=edge_index.dtype)
    src = jnp.concatenate([edge_index[0], loop])
    dst = jnp.concatenate([edge_index[1], loop])
    h = jax.nn.elu(_gat(x, src, dst, W1, a_src1, a_dst1, b1, n))
    h = jax.nn.elu(_gat(h, src, dst, W2, a_src2, a_dst2, b2, n))
    h = jax.nn.elu(_gat(h, src, dst, W3, a_src3, a_dst3, b3, n))
    h = jax.nn.elu(_gcn(h, src, dst, Wg1, bg1, n))
    h = jax.nn.elu(_gcn(h, src, dst, Wg2, bg2, n))
    h = _softmax_aggr(h, batch, t, G)
    out = pl.pallas_call(
        _mlp_body,
        out_shape=jax.ShapeDtypeStruct((G, 2), jnp.float32),
    )(h, Wl1, bl1.reshape(1, 16), Wl2, bl2.reshape(1, 2))
    return out


# speedup vs baseline: 20.8645x; 20.8645x over previous
import functools

import jax
import jax.numpy as jnp
from jax import lax
from jax.experimental import pallas as pl
from jax.experimental.pallas import tpu as pltpu
from jax.experimental.pallas import tpu_sc as plsc

N = 100000
E = 1600000
G = 128

NC = 2
NS = 16
NW = NC * NS

BN = 1024
NB = 98
NPAD = BN * NB

SUB = 128
KR = 8
EBLK = SUB * KR
ETPAD = 52 * NW * EBLK
ET128 = ETPAD // SUB

mesh = plsc.VectorSubcoreMesh(
    core_axis_name="c", subcore_axis_name="s", num_cores=NC, num_subcores=NS)

f32 = jnp.float32
i32 = jnp.int32


def _elu(v):
    return jnp.where(v > 0, v, jnp.exp(jnp.minimum(v, 0.0)) - 1.0)


def _lane(vec, i):
    return lax.gather(
        vec, jnp.full((16, 1), i, i32),
        lax.GatherDimensionNumbers(offset_dims=(), collapsed_slice_dims=(0,),
                                   start_index_map=(0,)),
        (1,), mode=lax.GatherScatterMode.PROMISE_IN_BOUNDS)


def _edge_softmax_body(src_r, dst_r, ts_r, td_r, scal_r, z1_r,
                       ex_o, ssum_o,
                       src_v, dst_v, ts_v, td_v, ex_v, scal_v,
                       acc_sh, sem, sem2):
    c = lax.axis_index("c")
    s = lax.axis_index("s")
    w = c * NS + s
    slc = NPAD // NS

    pltpu.sync_copy(scal_r, scal_v)
    pltpu.sync_copy(z1_r.at[pl.ds(s * slc, slc)], acc_sh.at[pl.ds(s * slc, slc)])
    plsc.subcore_barrier()

    srow = scal_v[pl.ds(0, 16)]
    cs_v = _lane(srow, 0)
    cd_v = _lane(srow, 1)
    gm_v = _lane(srow, 2)

    rbase = w * (ETPAD // NW // SUB)

    def blk(j, carry):
        roff = rbase + j * KR
        pltpu.sync_copy(src_r.at[pl.ds(roff, KR), :], src_v)
        pltpu.sync_copy(dst_r.at[pl.ds(roff, KR), :], dst_v)
        cps = [pltpu.async_copy(ts_r.at[src_v.at[k]], ts_v.at[k], sem)
               for k in range(KR)]
        cpd = [pltpu.async_copy(td_r.at[dst_v.at[k]], td_v.at[k], sem2)
               for k in range(KR)]
        for cp in cps + cpd:
            cp.wait()

        def chunk(q, carry2):
            k = q // 8
            t = q % 8
            a = ts_v[k, pl.ds(t * 16, 16)] * cs_v
            b = td_v[k, pl.ds(t * 16, 16)] * cd_v
            e = a + b
            e = jnp.where(e > 0, e, 0.2 * e)
            m = gm_v + b
            m = jnp.where(m > 0, m, 0.2 * m)
            ex_v[k, pl.ds(t * 16, 16)] = jnp.exp(e - m)
            return carry2

        lax.fori_loop(0, KR * 8, chunk, 0)
        pltpu.sync_copy(ex_v, ex_o.at[pl.ds(roff, KR), :])
        for k in range(KR):
            pltpu.sync_copy(ex_v.at[k], acc_sh.at[dst_v.at[k]], add=True)
        return carry

    lax.fori_loop(0, ETPAD // NW // EBLK, blk, 0)
    plsc.subcore_barrier()
    pltpu.sync_copy(acc_sh.at[pl.ds(s * slc, slc)],
                    ssum_o.at[c, pl.ds(s * slc, slc)])


_edge_softmax = functools.partial(
    pl.kernel,
    out_type=[jax.ShapeDtypeStruct((ET128, SUB), f32),
              jax.ShapeDtypeStruct((NC, NPAD), f32)],
    mesh=mesh,
    compiler_params=pltpu.CompilerParams(use_tc_tiling_on_sc=False),
    scratch_types=[
        pltpu.VMEM((KR, SUB), i32), pltpu.VMEM((KR, SUB), i32),
        pltpu.VMEM((KR, SUB), f32), pltpu.VMEM((KR, SUB), f32),
        pltpu.VMEM((KR, SUB), f32), pltpu.VMEM((16,), f32),
        pltpu.VMEM_SHARED((NPAD,), f32),
        pltpu.SemaphoreType.DMA, pltpu.SemaphoreType.DMA,
    ])(_edge_softmax_body)


def _scalar_agg_body(src_r, dst_r, ex_r, xt_r, z1_r,
                     s_o, deg_o,
                     src_v, dst_v, xs_v, ex_v, one_v,
                     s_sh, d_sh, sem):
    c = lax.axis_index("c")
    s = lax.axis_index("s")
    w = c * NS + s
    slc = NPAD // NS

    pltpu.sync_copy(z1_r.at[pl.ds(s * slc, slc)], s_sh.at[pl.ds(s * slc, slc)])
    pltpu.sync_copy(z1_r.at[pl.ds(s * slc, slc)], d_sh.at[pl.ds(s * slc, slc)])

    def fill(q, carry):
        one_v[pl.ds(q * 16, 16)] = jnp.zeros((16,), f32) + 1.0
        return carry

    lax.fori_loop(0, SUB // 16, fill, 0)
    plsc.subcore_barrier()

    rbase = w * (ETPAD // NW // SUB)

    def blk(j, carry):
        roff = rbase + j * KR
        pltpu.sync_copy(src_r.at[pl.ds(roff, KR), :], src_v)
        pltpu.sync_copy(dst_r.at[pl.ds(roff, KR), :], dst_v)
        pltpu.sync_copy(ex_r.at[pl.ds(roff, KR), :], ex_v)
        cps = [pltpu.async_copy(xt_r.at[src_v.at[k]], xs_v.at[k], sem)
               for k in range(KR)]
        for cp in cps:
            cp.wait()

        def chunk(q, carry2):
            k = q // 8
            t = q % 8
            xs_v[k, pl.ds(t * 16, 16)] = (
                xs_v[k, pl.ds(t * 16, 16)] * ex_v[k, pl.ds(t * 16, 16)])
            return carry2

        lax.fori_loop(0, KR * 8, chunk, 0)
        for k in range(KR):
            pltpu.sync_copy(xs_v.at[k], s_sh.at[dst_v.at[k]], add=True)
            pltpu.sync_copy(one_v, d_sh.at[dst_v.at[k]], add=True)
        return carry

    lax.fori_loop(0, ETPAD // NW // EBLK, blk, 0)
    plsc.subcore_barrier()
    pltpu.sync_copy(s_sh.at[pl.ds(s * slc, slc)],
                    s_o.at[c, pl.ds(s * slc, slc)])
    pltpu.sync_copy(d_sh.at[pl.ds(s * slc, slc)],
                    deg_o.at[c, pl.ds(s * slc, slc)])


_scalar_agg = functools.partial(
    pl.kernel,
    out_type=[jax.ShapeDtypeStruct((NC, NPAD), f32),
              jax.ShapeDtypeStruct((NC, NPAD), f32)],
    mesh=mesh,
    compiler_params=pltpu.CompilerParams(use_tc_tiling_on_sc=False),
    scratch_types=[
        pltpu.VMEM((KR, SUB), i32), pltpu.VMEM((KR, SUB), i32),
        pltpu.VMEM((KR, SUB), f32), pltpu.VMEM((KR, SUB), f32),
        pltpu.VMEM((SUB,), f32),
        pltpu.VMEM_SHARED((NPAD,), f32), pltpu.VMEM_SHARED((NPAD,), f32),
        pltpu.SemaphoreType.DMA,
    ])(_scalar_agg_body)


def _make_spmm(n_slabs, scaled):
    spc = n_slabs // NC

    def body(*refs):
        if scaled:
            (src_r, dst_r, ex_r, tab_r, z16_r, raw_o,
             src_v, dst_v, idx_v, ex_v, rows_v, acc_sh, sem) = refs
        else:
            (src_r, dst_r, tab_r, z16_r, raw_o,
             src_v, dst_v, idx_v, ex_v, rows_v, acc_sh, sem) = refs
            ex_r = None
        c = lax.axis_index("c")
        s = lax.axis_index("s")
        slc = NPAD // NS
        rbase = s * (ETPAD // NS // SUB)

        for slab_i in range(spc):
            slab = c * spc + slab_i
            pltpu.sync_copy(z16_r.at[pl.ds(s * slc, slc), :],
                            acc_sh.at[pl.ds(s * slc, slc), :])
            plsc.subcore_barrier()

            def blk(j, carry):
                roff = rbase + j * KR
                pltpu.sync_copy(src_r.at[pl.ds(roff, KR), :], src_v)
                pltpu.sync_copy(dst_r.at[pl.ds(roff, KR), :], dst_v)
                if scaled:
                    pltpu.sync_copy(ex_r.at[pl.ds(roff, KR), :], ex_v)

                def offs(q, carry2):
                    k = q // 8
                    t = q % 8
                    idx_v[k, pl.ds(t * 16, 16)] = (
                        src_v[k, pl.ds(t * 16, 16)] + slab * NPAD)
                    return carry2

                lax.fori_loop(0, KR * 8, offs, 0)
                cps = [pltpu.async_copy(tab_r.at[idx_v.at[k]], rows_v.at[k], sem)
                       for k in range(KR)]
                for cp in cps:
                    cp.wait()
                if scaled:
                    for k in range(KR):
                        def scale_chunk(cc, carry2, k=k):
                            exc = ex_v[k, pl.ds(cc * 16, 16)]
                            for i in range(16):
                                r = cc * 16 + i
                                rows_v[k, r, :] = (
                                    rows_v[k, r, :] * _lane(exc, i))
                            return carry2

                        lax.fori_loop(0, SUB // 16, scale_chunk, 0)
                for k in range(KR):
                    pltpu.sync_copy(rows_v.at[k], acc_sh.at[dst_v.at[k]],
                                    add=True)
                return carry

            lax.fori_loop(0, ETPAD // NS // EBLK, blk, 0)
            plsc.subcore_barrier()
            pltpu.sync_copy(
                acc_sh.at[pl.ds(s * slc, slc), :],
                raw_o.at[pl.ds(slab * NPAD + s * slc, slc), :])

    scratch = [
        pltpu.VMEM((KR, SUB), i32), pltpu.VMEM((KR, SUB), i32),
        pltpu.VMEM((KR, SUB), i32), pltpu.VMEM((KR, SUB), f32),
        pltpu.VMEM((KR, SUB, 16), f32),
        pltpu.VMEM_SHARED((NPAD, 16), f32),
        pltpu.SemaphoreType.DMA,
    ]
    return functools.partial(
        pl.kernel,
        out_type=[jax.ShapeDtypeStruct((n_slabs * NPAD, 16), f32)],
        mesh=mesh,
        compiler_params=pltpu.CompilerParams(use_tc_tiling_on_sc=False),
        scratch_types=scratch)(body)


_spmm_gat = _make_spmm(8, True)
_spmm_gcn1 = _make_spmm(4, False)
_spmm_gcn2 = _make_spmm(2, False)


def _lane_iota():
    return jax.lax.broadcasted_iota(i32, (1, 128), 1)


def _scalars0_body(x_ref, W1_ref, as_ref, ad_ref, out_ref, mx, mn):
    i = pl.program_id(0)
    xb = x_ref[...]
    bmax = jnp.max(xb)
    bmin = jnp.min(xb)

    @pl.when(i == 0)
    def _():
        mx[0, 0] = bmax
        mn[0, 0] = bmin

    @pl.when(i > 0)
    def _():
        mx[0, 0] = jnp.maximum(mx[0, 0], bmax)
        mn[0, 0] = jnp.minimum(mn[0, 0], bmin)

    @pl.when(i == NB - 1)
    def _():
        cs = jnp.sum(W1_ref[...] * as_ref[...])
        cd = jnp.sum(W1_ref[...] * ad_ref[...])
        gm = jnp.where(cs >= 0, cs * mx[0, 0], cs * mn[0, 0])
        lane = _lane_iota()
        out_ref[...] = jnp.where(
            lane == 0, cs,
            jnp.where(lane == 1, cd, jnp.where(lane == 2, gm, 0.0)))


def _tc_scalars0(x2d, W1, as1, ad1):
    return pl.pallas_call(
        _scalars0_body,
        grid=(NB,),
        in_specs=[
            pl.BlockSpec((BN, 1), lambda i: (i, 0)),
            pl.BlockSpec((1, 128), lambda i: (0, 0)),
            pl.BlockSpec((1, 128), lambda i: (0, 0)),
            pl.BlockSpec((1, 128), lambda i: (0, 0)),
        ],
        out_specs=pl.BlockSpec((1, 128), lambda i: (0, 0)),
        out_shape=jax.ShapeDtypeStruct((1, 128), f32),
        scratch_shapes=[pltpu.SMEM((1, 1), f32), pltpu.SMEM((1, 1), f32)],
    )(x2d, W1, as1, ad1)


def _prep_common(h, W, as_ref, ad_ref, hw_ref, asv_ref, adv_ref, scal_ref, mx,
                 i):
    hw = jnp.dot(h, W[...], preferred_element_type=f32)
    hw_ref[...] = hw
    asv = jnp.dot(hw, as_ref[...], preferred_element_type=f32)
    adv = jnp.dot(hw, ad_ref[...], preferred_element_type=f32)
    asv_ref[...] = asv
    adv_ref[...] = adv
    bmax = jnp.max(asv)

    @pl.when(i == 0)
    def _():
        mx[0, 0] = bmax

    @pl.when(i > 0)
    def _():
        mx[0, 0] = jnp.maximum(mx[0, 0], bmax)

    @pl.when(i == NB - 1)
    def _():
        lane = _lane_iota()
        scal_ref[...] = jnp.where(
            lane == 0, 1.0,
            jnp.where(lane == 1, 1.0, jnp.where(lane == 2, mx[0, 0], 0.0)))


def _prep2_body(s0, s1, ss0, ss1, W1_ref, b1_ref, W2_ref, as_ref, ad_ref,
                hw_ref, asv_ref, adv_ref, scal_ref, mx):
    i = pl.program_id(0)
    ssum = ss0[...] + ss1[...]
    o1 = (s0[...] + s1[...]) / (ssum + 1e-16)
    h = _elu(jnp.dot(o1, W1_ref[...], preferred_element_type=f32) + b1_ref[...])
    _prep_common(h, W2_ref, as_ref, ad_ref, hw_ref, asv_ref, adv_ref,
                 scal_ref, mx, i)


def _prep3_body(raw, ss0, ss1, b_ref, W_ref, as_ref, ad_ref,
                hw_ref, asv_ref, adv_ref, scal_ref, mx):
    i = pl.program_id(0)
    ssum = ss0[...] + ss1[...]
    o = raw[...] / (ssum + 1e-16)
    h = _elu(o + b_ref[...])
    _prep_common(h, W_ref, as_ref, ad_ref, hw_ref, asv_ref, adv_ref,
                 scal_ref, mx, i)


def _col(i):
    return pl.BlockSpec((BN, 1), lambda i_: (i_, 0))


def _full(r, c):
    return pl.BlockSpec((r, c), lambda i_: (0, 0))


def _tc_prep2(s_part, ssum_part, W1, b1, W2, as2, ad2):
    s0 = s_part[0].reshape(NPAD, 1)
    s1 = s_part[1].reshape(NPAD, 1)
    ss0 = ssum_part[0].reshape(NPAD, 1)
    ss1 = ssum_part[1].reshape(NPAD, 1)
    return pl.pallas_call(
        _prep2_body,
        grid=(NB,),
        in_specs=[_col(0), _col(0), _col(0), _col(0),
                  _full(1, 128), _full(1, 128), _full(128, 128),
                  pl.BlockSpec((128, 1), lambda i: (0, 0)),
                  pl.BlockSpec((128, 1), lambda i: (0, 0))],
        out_specs=[pl.BlockSpec((BN, 128), lambda i: (i, 0)),
                   _col(0), _col(0), _full(1, 128)],
        out_shape=[jax.ShapeDtypeStruct((NPAD, 128), f32),
                   jax.ShapeDtypeStruct((NPAD, 1), f32),
                   jax.ShapeDtypeStruct((NPAD, 1), f32),
                   jax.ShapeDtypeStruct((1, 128), f32)],
        scratch_shapes=[pltpu.SMEM((1, 1), f32)],
    )(s0, s1, ss0, ss1, W1, b1, W2, as2.reshape(128, 1), ad2.reshape(128, 1))


def _tc_prep3(raw, ssum_part, b_prev, W, a_s, a_d):
    ss0 = ssum_part[0].reshape(NPAD, 1)
    ss1 = ssum_part[1].reshape(NPAD, 1)
    return pl.pallas_call(
        _prep3_body,
        grid=(NB,),
        in_specs=[pl.BlockSpec((BN, 128), lambda i: (i, 0)),
                  _col(0), _col(0),
                  _full(1, 128), _full(128, 128),
                  pl.BlockSpec((128, 1), lambda i: (0, 0)),
                  pl.BlockSpec((128, 1), lambda i: (0, 0))],
        out_specs=[pl.BlockSpec((BN, 128), lambda i: (i, 0)),
                   _col(0), _col(0), _full(1, 128)],
        out_shape=[jax.ShapeDtypeStruct((NPAD, 128), f32),
                   jax.ShapeDtypeStruct((NPAD, 1), f32),
                   jax.ShapeDtypeStruct((NPAD, 1), f32),
                   jax.ShapeDtypeStruct((1, 128), f32)],
        scratch_shapes=[pltpu.SMEM((1, 1), f32)],
    )(raw, ss0, ss1, b_prev, W, a_s.reshape(128, 1), a_d.reshape(128, 1))


def _prep_g1_body(raw, ss0, ss1, d0, d1, b_ref, W_ref, hw_ref, dis_ref):
    ssum = ss0[...] + ss1[...]
    o = raw[...] / (ssum + 1e-16)
    h = _elu(o + b_ref[...])
    deg = d0[...] + d1[...]
    dis = jnp.where(deg > 0, jax.lax.rsqrt(jnp.maximum(deg, 1e-30)), 0.0)
    dis_ref[...] = dis
    hw_ref[...] = jnp.dot(h, W_ref[...], preferred_element_type=f32) * dis


def _tc_prep_g1(raw, ssum_part, deg_part, b3, Wg1):
    ss0 = ssum_part[0].reshape(NPAD, 1)
    ss1 = ssum_part[1].reshape(NPAD, 1)
    d0 = deg_part[0].reshape(NPAD, 1)
    d1 = deg_part[1].reshape(NPAD, 1)
    return pl.pallas_call(
        _prep_g1_body,
        grid=(NB,),
        in_specs=[pl.BlockSpec((BN, 128), lambda i: (i, 0)),
                  _col(0), _col(0), _col(0), _col(0),
                  _full(1, 128), _full(128, 64)],
        out_specs=[pl.BlockSpec((BN, 64), lambda i: (i, 0)), _col(0)],
        out_shape=[jax.ShapeDtypeStruct((NPAD, 64), f32),
                   jax.ShapeDtypeStruct((NPAD, 1), f32)],
    )(raw, ss0, ss1, d0, d1, b3, Wg1)


def _prep_g2_body(raw, dis_ref, b_ref, W_ref, hw_ref):
    o = raw[...] * dis_ref[...]
    h = _elu(o + b_ref[...])
    hw_ref[...] = jnp.dot(h, W_ref[...], preferred_element_type=f32) * dis_ref[...]


def _tc_prep_g2(raw, dis, bg1, Wg2):
    return pl.pallas_call(
        _prep_g2_body,
        grid=(NB,),
        in_specs=[pl.BlockSpec((BN, 64), lambda i: (i, 0)), _col(0),
                  _full(1, 64), _full(64, 32)],
        out_specs=pl.BlockSpec((BN, 32), lambda i: (i, 0)),
        out_shape=jax.ShapeDtypeStruct((NPAD, 32), f32),
    )(raw, dis, bg1, Wg2)


def _final_body(raw, dis_ref, b_ref, batch_ref, t_ref,
                Wl1_ref, bl1_ref, Wl2_ref, bl2_ref, out_ref,
                mv, t1, t2):
    p = pl.program_id(0)
    i = pl.program_id(1)
    o = raw[...] * dis_ref[...]
    x6 = _elu(o + b_ref[...])
    sc = x6 * t_ref[0, 0]

    @pl.when(p == 0)
    def _():
        bm = jnp.max(sc, axis=0, keepdims=True)

        @pl.when(i == 0)
        def _():
            mv[...] = bm

        @pl.when(i > 0)
        def _():
            mv[...] = jnp.maximum(mv[...], bm)

    @pl.when(p == 1)
    def _():
        @pl.when(i == 0)
        def _():
            t1[...] = jnp.zeros_like(t1)
            t2[...] = jnp.zeros_like(t2)

        ex = jnp.exp(sc - mv[...])
        bflat = batch_ref[...].reshape(1, BN)
        oh = (jax.lax.broadcasted_iota(i32, (G, BN), 0)
              == jnp.broadcast_to(bflat, (G, BN))).astype(f32)
        t1[...] += jnp.dot(oh, ex, preferred_element_type=f32)
        t2[...] += jnp.dot(oh, ex * x6, preferred_element_type=f32)

        @pl.when(i == NB - 1)
        def _():
            res = t2[...] / (t1[...] + 1e-16)
            m1 = _elu(jnp.dot(res, Wl1_ref[...], preferred_element_type=f32)
                      + bl1_ref[...])
            ho = jnp.dot(m1, Wl2_ref[...], preferred_element_type=f32) + bl2_ref[...]
            rmax = jnp.max(ho, axis=1, keepdims=True)
            lse = jnp.log(jnp.sum(jnp.exp(ho - rmax), axis=1, keepdims=True)) + rmax
            out_ref[...] = ho - lse


def _tc_final(raw, dis, bg2, batch3, t2d, Wl1, bl1, Wl2, bl2):
    return pl.pallas_call(
        _final_body,
        grid=(2, NB),
        in_specs=[pl.BlockSpec((BN, 32), lambda p, i: (i, 0)),
                  pl.BlockSpec((BN, 1), lambda p, i: (i, 0)),
                  pl.BlockSpec((1, 32), lambda p, i: (0, 0)),
                  pl.BlockSpec((1, 8, 128), lambda p, i: (i, 0, 0)),
                  pl.BlockSpec((1, 1), lambda p, i: (0, 0)),
                  pl.BlockSpec((32, 16), lambda p, i: (0, 0)),
                  pl.BlockSpec((1, 16), lambda p, i: (0, 0)),
                  pl.BlockSpec((16, 2), lambda p, i: (0, 0)),
                  pl.BlockSpec((1, 2), lambda p, i: (0, 0))],
        out_specs=pl.BlockSpec((G, 2), lambda p, i: (0, 0)),
        out_shape=jax.ShapeDtypeStruct((G, 2), f32),
        scratch_shapes=[pltpu.VMEM((1, 32), f32),
                        pltpu.VMEM((G, 32), f32),
                        pltpu.VMEM((G, 32), f32)],
    )(raw, dis, bg2, batch3, t2d, Wl1, bl1, Wl2, bl2)


def _slab_major(hw, d):
    return hw.reshape(NPAD, d // 16, 16).transpose(1, 0, 2).reshape(-1, 16)


def _from_slab(raw, d):
    return raw.reshape(d // 16, NPAD, 16).transpose(1, 0, 2).reshape(NPAD, d)


def kernel(x, edge_index, batch, W1, a_src1, a_dst1, b1, W2, a_src2, a_dst2,
           b2, W3, a_src3, a_dst3, b3, Wg1, bg1, Wg2, bg2, t, Wl1, bl1, Wl2,
           bl2):
    loop = jnp.arange(N, dtype=jnp.int32)
    pad_e = ETPAD - E - N
    src = jnp.concatenate([edge_index[0], loop,
                           jnp.full((pad_e,), N, jnp.int32)]).reshape(ET128, SUB)
    dst = jnp.concatenate([edge_index[1], loop,
                           jnp.full((pad_e,), N, jnp.int32)]).reshape(ET128, SUB)

    xf = jnp.pad(x[:, 0], (0, NPAD - N))
    z1 = jnp.zeros((NPAD,), f32)
    z16 = jnp.zeros((NPAD, 16), f32)

    scal1 = _tc_scalars0(xf.reshape(NPAD, 1), W1,
                         a_src1.reshape(1, 128), a_dst1.reshape(1, 128))[0, :16]
    ex1, ssum1 = _edge_softmax(src, dst, xf, xf, scal1, z1)
    s_part, deg_part = _scalar_agg(src, dst, ex1, xf, z1)

    hw2, asv2, adv2, scal2 = _tc_prep2(s_part, ssum1, W1, b1.reshape(1, 128),
                                       W2, a_src2, a_dst2)
    ex2, ssum2 = _edge_softmax(src, dst, asv2.reshape(NPAD), adv2.reshape(NPAD),
                               scal2[0, :16], z1)
    raw2 = _spmm_gat(src, dst, ex2, _slab_major(hw2, 128), z16)[0]

    hw3, asv3, adv3, scal3 = _tc_prep3(_from_slab(raw2, 128), ssum2,
                                       b2.reshape(1, 128), W3, a_src3, a_dst3)
    ex3, ssum3 = _edge_softmax(src, dst, asv3.reshape(NPAD), adv3.reshape(NPAD),
                               scal3[0, :16], z1)
    raw3 = _spmm_gat(src, dst, ex3, _slab_major(hw3, 128), z16)[0]

    hwg1, dis = _tc_prep_g1(_from_slab(raw3, 128), ssum3, deg_part,
                            b3.reshape(1, 128), Wg1)
    raw4 = _spmm_gcn1(src, dst, _slab_major(hwg1, 64), z16)[0]

    hwg2 = _tc_prep_g2(_from_slab(raw4, 64), dis, bg1.reshape(1, 64), Wg2)
    raw5 = _spmm_gcn2(src, dst, _slab_major(hwg2, 32), z16)[0]

    batch3 = jnp.pad(batch, (0, NPAD - N), constant_values=G).reshape(NB, 8, 128)
    out = _tc_final(_from_slab(raw5, 32), dis, bg2.reshape(1, 32), batch3,
                    t.reshape(1, 1), Wl1, bl1.reshape(1, 16), Wl2,
                    bl2.reshape(1, 2))
    return out
